# baseline (speedup 1.0000x reference)
import functools
import math

import jax
import jax.numpy as jnp
from jax import lax
from jax.experimental import pallas as pl
from jax.experimental.pallas import tpu as pltpu
from jax.experimental.pallas import tpu_sc as plsc

NUM_CODEBOOKS = 4
VOCAB = 100000
D = 128
B, S = 1024, 200
N = B * S
SCALE = math.sqrt(D)

NC, NS, L = 2, 16, 16
NW = NC * NS
B_PER_W = N // NW
CHUNK = 80
NCHUNK = B_PER_W // CHUNK

TBLK = 1000


def _transform_tables(w0, w1, w2, w3, comb_w, comb_b2d):
    def body(w0_ref, w1_ref, w2_ref, w3_ref, cw_ref, cb_ref,
             t0_ref, t1_ref, t2_ref, t3_ref):
        c = cw_ref[...]
        for i, (w_ref, t_ref) in enumerate(
                zip((w0_ref, w1_ref, w2_ref, w3_ref),
                    (t0_ref, t1_ref, t2_ref, t3_ref))):
            acc = jnp.dot(w_ref[...], c[i * D:(i + 1) * D, :],
                          preferred_element_type=jnp.float32) * SCALE
            if i == 0:
                acc = acc + cb_ref[...] * SCALE
            t_ref[...] = acc

    tbl_spec = pl.BlockSpec((TBLK, D), lambda r: (r, 0))
    return pl.pallas_call(
        body,
        grid=(VOCAB // TBLK,),
        in_specs=[tbl_spec] * 4 + [
            pl.BlockSpec((NUM_CODEBOOKS * D, D), lambda r: (0, 0)),
            pl.BlockSpec((1, D), lambda r: (0, 0)),
        ],
        out_specs=[tbl_spec] * 4,
        out_shape=[jax.ShapeDtypeStruct((VOCAB, D), jnp.float32)] * 4,
    )(w0, w1, w2, w3, comb_w, comb_b2d)


def _gather_sum(idx_flat, t0, t1, t2, t3):
    mesh = plsc.VectorSubcoreMesh(core_axis_name="c", subcore_axis_name="s")

    scratch = (
        [pltpu.VMEM((B_PER_W,), jnp.int32)] * 4
        + [pltpu.VMEM((CHUNK, D), jnp.float32)] * 8
        + [pltpu.VMEM((CHUNK, D), jnp.float32)] * 2
        + [pltpu.SemaphoreType.DMA] * 8
        + [pltpu.SemaphoreType.DMA] * 2
    )

    @functools.partial(
        pl.kernel,
        mesh=mesh,
        out_type=jax.ShapeDtypeStruct((N, D), jnp.float32),
        scratch_types=scratch,
    )
    def k(idx_hbm, t0_hbm, t1_hbm, t2_hbm, t3_hbm, out_hbm,
          ix0, ix1, ix2, ix3,
          ba0, ba1, ba2, ba3, bb0, bb1, bb2, bb3, ra, rb,
          sa0, sa1, sa2, sa3, sb0, sb1, sb2, sb3, oa, ob):
        wid = lax.axis_index("s") * NC + lax.axis_index("c")
        cbase = wid * NCHUNK
        ixs = (ix0, ix1, ix2, ix3)
        tbls = (t0_hbm, t1_hbm, t2_hbm, t3_hbm)
        bufs = ((ba0, ba1, ba2, ba3), (bb0, bb1, bb2, bb3))
        res = (ra, rb)
        gsems = ((sa0, sa1, sa2, sa3), (sb0, sb1, sb2, sb3))
        osems = (oa, ob)
        for i in range(NUM_CODEBOOKS):
            pltpu.sync_copy(
                idx_hbm.at[pl.ds(i * N + wid * B_PER_W, B_PER_W)], ixs[i])

        HALF = CHUNK // 2

        def gather_cp(ch, p, i, h):
            return pltpu.make_async_copy(
                tbls[i].at[ixs[i].at[pl.ds(ch * CHUNK + h * HALF, HALF)]],
                bufs[p][i].at[pl.ds(h * HALF, HALF)], gsems[p][i])

        def out_cp(ch, p):
            return pltpu.make_async_copy(
                res[p], out_hbm.at[pl.ds((cbase + ch) * CHUNK, CHUNK)],
                osems[p])

        def fire(ch, p):
            for i in range(NUM_CODEBOOKS):
                for h in range(2):
                    gather_cp(ch, p, i, h).start()

        fire(0, 0)

        def pair_body(j, _):
            for p in range(2):
                ch = 2 * j + p
                nxt = ch + 1
                if p == 0:
                    fire(nxt, 1)
                else:
                    @pl.when(j < NCHUNK // 2 - 1)
                    def _():
                        fire(nxt, 0)
                for i in range(NUM_CODEBOOKS):
                    for h in range(2):
                        gather_cp(ch, p, i, h).wait()

                @pl.when(j >= 1)
                def _():
                    out_cp(ch, p).wait()

                b0, b1, b2, b3 = bufs[p]
                r = res[p]

                def row_body(row, _):
                    for g in range(D // L):
                        sl = pl.ds(g * L, L)
                        r[row, sl] = (b0[row, sl] + b1[row, sl]) + (
                            b2[row, sl] + b3[row, sl])
                    return 0

                lax.fori_loop(0, CHUNK, row_body, 0, unroll=False)
                out_cp(ch, p).start()
            return 0

        lax.fori_loop(0, NCHUNK // 2, pair_body, 0, unroll=False)
        for p in range(2):
            out_cp(NCHUNK - 2 + p, p).wait()

    return k(idx_flat, t0, t1, t2, t3)


def kernel(tokens, W0, W1, W2, W3, comb_W, comb_b):
    t0, t1, t2, t3 = _transform_tables(
        W0, W1, W2, W3, comb_W, comb_b.reshape(1, D))
    idx_flat = (
        tokens.astype(jnp.int32)
        .reshape(N, NUM_CODEBOOKS)
        .T.reshape(NUM_CODEBOOKS * N)
    )
    out = _gather_sum(idx_flat, t0, t1, t2, t3)
    return out.reshape(B, S, D)


# speedup vs baseline: 1.1071x; 1.1071x over previous
import functools
import math

import jax
import jax.numpy as jnp
from jax import lax
from jax.experimental import pallas as pl
from jax.experimental.pallas import tpu as pltpu
from jax.experimental.pallas import tpu_sc as plsc

NUM_CODEBOOKS = 4
VOCAB = 100000
D = 128
B, S = 1024, 200
N = B * S
SCALE = math.sqrt(D)

NC, NS, L = 2, 16, 16
NW = NC * NS
B_PER_W = N // NW
CHUNK = 80
NCHUNK = B_PER_W // CHUNK

TBLK = 5000


def _transform_tables(w0, w1, w2, w3, comb_w, comb_b2d):
    def body(w0_ref, w1_ref, w2_ref, w3_ref, cw_ref, cb_ref,
             t0_ref, t1_ref, t2_ref, t3_ref):
        c = cw_ref[...]
        for i, (w_ref, t_ref) in enumerate(
                zip((w0_ref, w1_ref, w2_ref, w3_ref),
                    (t0_ref, t1_ref, t2_ref, t3_ref))):
            acc = jnp.dot(w_ref[...], c[i * D:(i + 1) * D, :],
                          preferred_element_type=jnp.float32) * SCALE
            if i == 0:
                acc = acc + cb_ref[...] * SCALE
            t_ref[...] = acc

    tbl_spec = pl.BlockSpec((TBLK, D), lambda r: (r, 0))
    return pl.pallas_call(
        body,
        grid=(VOCAB // TBLK,),
        in_specs=[tbl_spec] * 4 + [
            pl.BlockSpec((NUM_CODEBOOKS * D, D), lambda r: (0, 0)),
            pl.BlockSpec((1, D), lambda r: (0, 0)),
        ],
        out_specs=[tbl_spec] * 4,
        out_shape=[jax.ShapeDtypeStruct((VOCAB, D), jnp.float32)] * 4,
    )(w0, w1, w2, w3, comb_w, comb_b2d)


def _gather_sum(idx_flat, t0, t1, t2, t3):
    mesh = plsc.VectorSubcoreMesh(core_axis_name="c", subcore_axis_name="s")

    scratch = (
        [pltpu.VMEM((B_PER_W,), jnp.int32)] * 4
        + [pltpu.VMEM((CHUNK, D), jnp.float32)] * 8
        + [pltpu.VMEM((CHUNK, D), jnp.float32)] * 2
        + [pltpu.SemaphoreType.DMA] * 8
        + [pltpu.SemaphoreType.DMA] * 2
    )

    @functools.partial(
        pl.kernel,
        mesh=mesh,
        out_type=jax.ShapeDtypeStruct((N, D), jnp.float32),
        scratch_types=scratch,
    )
    def k(idx_hbm, t0_hbm, t1_hbm, t2_hbm, t3_hbm, out_hbm,
          ix0, ix1, ix2, ix3,
          ba0, ba1, ba2, ba3, bb0, bb1, bb2, bb3, ra, rb,
          sa0, sa1, sa2, sa3, sb0, sb1, sb2, sb3, oa, ob):
        wid = lax.axis_index("s") * NC + lax.axis_index("c")
        cbase = wid * NCHUNK
        ixs = (ix0, ix1, ix2, ix3)
        tbls = (t0_hbm, t1_hbm, t2_hbm, t3_hbm)
        bufs = ((ba0, ba1, ba2, ba3), (bb0, bb1, bb2, bb3))
        res = (ra, rb)
        gsems = ((sa0, sa1, sa2, sa3), (sb0, sb1, sb2, sb3))
        osems = (oa, ob)
        for i in range(NUM_CODEBOOKS):
            pltpu.sync_copy(
                idx_hbm.at[pl.ds(i * N + wid * B_PER_W, B_PER_W)], ixs[i])

        HALF = CHUNK // 2

        def gather_cp(ch, p, i, h):
            return pltpu.make_async_copy(
                tbls[i].at[ixs[i].at[pl.ds(ch * CHUNK + h * HALF, HALF)]],
                bufs[p][i].at[pl.ds(h * HALF, HALF)], gsems[p][i])

        def out_cp(ch, p):
            return pltpu.make_async_copy(
                res[p], out_hbm.at[pl.ds((cbase + ch) * CHUNK, CHUNK)],
                osems[p])

        def fire(ch, p):
            for i in range(NUM_CODEBOOKS):
                for h in range(2):
                    gather_cp(ch, p, i, h).start()

        fire(0, 0)

        def pair_body(j, _):
            for p in range(2):
                ch = 2 * j + p
                nxt = ch + 1
                if p == 0:
                    fire(nxt, 1)
                else:
                    @pl.when(j < NCHUNK // 2 - 1)
                    def _():
                        fire(nxt, 0)
                for i in range(NUM_CODEBOOKS):
                    for h in range(2):
                        gather_cp(ch, p, i, h).wait()

                @pl.when(j >= 1)
                def _():
                    out_cp(ch, p).wait()

                b0, b1, b2, b3 = bufs[p]
                r = res[p]

                def row_body(row, _):
                    for g in range(D // L):
                        sl = pl.ds(g * L, L)
                        r[row, sl] = (b0[row, sl] + b1[row, sl]) + (
                            b2[row, sl] + b3[row, sl])
                    return 0

                lax.fori_loop(0, CHUNK, row_body, 0, unroll=False)
                out_cp(ch, p).start()
            return 0

        lax.fori_loop(0, NCHUNK // 2, pair_body, 0, unroll=False)
        for p in range(2):
            out_cp(NCHUNK - 2 + p, p).wait()

    return k(idx_flat, t0, t1, t2, t3)


def kernel(tokens, W0, W1, W2, W3, comb_W, comb_b):
    t0, t1, t2, t3 = _transform_tables(
        W0, W1, W2, W3, comb_W, comb_b.reshape(1, D))
    idx_flat = (
        tokens.astype(jnp.int32)
        .reshape(N, NUM_CODEBOOKS)
        .T.reshape(NUM_CODEBOOKS * N)
    )
    out = _gather_sum(idx_flat, t0, t1, t2, t3)
    return out.reshape(B, S, D)
